# baseline (speedup 1.0000x reference)
import jax
import jax.numpy as jnp
from jax.experimental import pallas as pl
from jax.experimental.pallas import tpu as pltpu

N = 100000
D = 128
K = 16
OUT = 256
BN1 = 10000
BN2 = 10000
NB1 = N // BN1
NB2 = N // BN2


def _reduce_kernel(ev_ref, x_ref, evals_ref, w1_ref, b1_ref, w2_ref, b2_ref,
                   mask_ref, wp_ref, wc_ref, acc_ref):
    i = pl.program_id(0)

    @pl.when(i == 0)
    def _init():
        acc_ref[...] = jnp.zeros_like(acc_ref)

    acc_ref[...] += jax.lax.dot_general(
        ev_ref[...], x_ref[...], (((0,), (0,)), ((), ())),
        preferred_element_type=jnp.float32)

    @pl.when(i == NB1 - 1)
    def _finalize():
        h = jax.lax.dot_general(
            w1_ref[...], evals_ref[...], (((0,), (0,)), ((), ())),
            preferred_element_type=jnp.float32)
        h = jnp.maximum(h + b1_ref[...], 0.0)
        f = jax.lax.dot_general(
            w2_ref[...], h, (((0,), (0,)), ((), ())),
            preferred_element_type=jnp.float32)
        f = jnp.tanh(f + b2_ref[...]) * mask_ref[...]
        x_filt = f * acc_ref[...]
        wc_ref[...] = jnp.dot(x_filt, wp_ref[...],
                              preferred_element_type=jnp.float32)


def _stream_kernel(ev_ref, wc_ref, bp_ref, out_ref):
    out_ref[...] = jnp.dot(ev_ref[...], wc_ref[...],
                           preferred_element_type=jnp.float32) + bp_ref[...]


def kernel(x, eigvecs, eigvals, eig_mask, W1, b1, W2, b2, Wp, bp):
    evals_c = eigvals.reshape(K, 1)
    b1_c = b1.reshape(K // 2, 1)
    b2_c = b2.reshape(K, 1)
    mask_c = eig_mask.astype(jnp.float32).reshape(K, 1)
    bp_r = bp.reshape(1, OUT)

    w_comb = pl.pallas_call(
        _reduce_kernel,
        grid=(NB1,),
        in_specs=[
            pl.BlockSpec((BN1, K), lambda i: (i, 0)),
            pl.BlockSpec((BN1, D), lambda i: (i, 0)),
            pl.BlockSpec((K, 1), lambda i: (0, 0)),
            pl.BlockSpec((K, K // 2), lambda i: (0, 0)),
            pl.BlockSpec((K // 2, 1), lambda i: (0, 0)),
            pl.BlockSpec((K // 2, K), lambda i: (0, 0)),
            pl.BlockSpec((K, 1), lambda i: (0, 0)),
            pl.BlockSpec((K, 1), lambda i: (0, 0)),
            pl.BlockSpec((D, OUT), lambda i: (0, 0)),
        ],
        out_specs=pl.BlockSpec((K, OUT), lambda i: (0, 0)),
        out_shape=jax.ShapeDtypeStruct((K, OUT), jnp.float32),
        scratch_shapes=[pltpu.VMEM((K, D), jnp.float32)],
    )(eigvecs, x, evals_c, W1, b1_c, W2, b2_c, mask_c, Wp)

    out = pl.pallas_call(
        _stream_kernel,
        grid=(NB2,),
        in_specs=[
            pl.BlockSpec((BN2, K), lambda i: (i, 0)),
            pl.BlockSpec((K, OUT), lambda i: (0, 0)),
            pl.BlockSpec((1, OUT), lambda i: (0, 0)),
        ],
        out_specs=pl.BlockSpec((BN2, OUT), lambda i: (i, 0)),
        out_shape=jax.ShapeDtypeStruct((N, OUT), jnp.float32),
    )(eigvecs, w_comb, bp_r)

    return out
